# Initial kernel scaffold; baseline (speedup 1.0000x reference)
import jax
import jax.numpy as jnp
from jax.experimental import pallas as pl


def kernel(block_input, raw_input, edge_attr, edge_index):
    raise NotImplementedError("write your pallas kernel here")


# speedup vs baseline: 1.4852x; 1.4852x over previous
import functools

import jax
import jax.numpy as jnp
from jax import lax
from jax.experimental import pallas as pl
from jax.experimental.pallas import tpu as pltpu
from jax.experimental.pallas import tpu_sc as plsc

N_NODES = 10000
N_EDGES = 320000
D_BLOCK = 128
D_EDGE = 16
D_OUT = D_EDGE + 2 * D_BLOCK
LANES = 16

NC = 2
NS = 16
NW = NC * NS

EDGES_PER_W = N_EDGES // NW
CHUNK = 80
N_CHUNKS = EDGES_PER_W // CHUNK


def _make_router():
    mesh = plsc.VectorSubcoreMesh(core_axis_name="c", subcore_axis_name="s")

    @functools.partial(
        pl.kernel,
        out_type=jax.ShapeDtypeStruct((N_EDGES, D_OUT), jnp.float32),
        mesh=mesh,
        scratch_types=[
            pltpu.VMEM((CHUNK,), jnp.int32),
            pltpu.VMEM((CHUNK,), jnp.int32),
            pltpu.VMEM((CHUNK, D_BLOCK), jnp.float32),
            pltpu.VMEM((CHUNK, D_BLOCK), jnp.float32),
            pltpu.VMEM((CHUNK, D_EDGE), jnp.float32),
            pltpu.VMEM((CHUNK, D_OUT), jnp.float32),
            pltpu.SemaphoreType.DMA,
            pltpu.SemaphoreType.DMA,
        ],
    )
    def router(tbl_hbm, attr_hbm, eidx_hbm, out_hbm,
               sidx_v, didx_v, srows_v, drows_v, attr_v, obuf_v, gsem1, gsem2):
        wid = lax.axis_index("s") * NC + lax.axis_index("c")
        base = wid * EDGES_PER_W

        def body(i, carry):
            off = base + i * CHUNK
            pltpu.sync_copy(eidx_hbm.at[pl.ds(off, CHUNK)], sidx_v)
            pltpu.sync_copy(eidx_hbm.at[pl.ds(N_EDGES + off, CHUNK)], didx_v)
            g1 = pltpu.async_copy(tbl_hbm.at[sidx_v], srows_v, gsem1)
            g2 = pltpu.async_copy(tbl_hbm.at[didx_v], drows_v, gsem2)
            pltpu.sync_copy(attr_hbm.at[pl.ds(off, CHUNK), :], attr_v)
            g1.wait()
            g2.wait()

            def fill(r, c):
                obuf_v[r, pl.ds(0, D_EDGE)] = attr_v[r, :]
                for j in range(D_BLOCK // LANES):
                    obuf_v[r, pl.ds(D_EDGE + j * LANES, LANES)] = (
                        srows_v[r, pl.ds(j * LANES, LANES)])
                    obuf_v[r, pl.ds(D_EDGE + D_BLOCK + j * LANES, LANES)] = (
                        drows_v[r, pl.ds(j * LANES, LANES)])
                return c

            lax.fori_loop(0, CHUNK, fill, 0)
            pltpu.sync_copy(obuf_v, out_hbm.at[pl.ds(off, CHUNK), :])
            return carry

        lax.fori_loop(0, N_CHUNKS, body, 0)

    return router


_router = _make_router()


def kernel(block_input, raw_input, edge_attr, edge_index):
    del raw_input
    eidx_flat = edge_index.astype(jnp.int32).reshape(-1)
    return _router(block_input, edge_attr, eidx_flat)


# speedup vs baseline: 2.1622x; 1.4558x over previous
import functools

import jax
import jax.numpy as jnp
from jax import lax
from jax.experimental import pallas as pl
from jax.experimental.pallas import tpu as pltpu
from jax.experimental.pallas import tpu_sc as plsc

N_NODES = 10000
N_EDGES = 320000
D_BLOCK = 128
D_EDGE = 16
D_OUT = D_EDGE + 2 * D_BLOCK
LANES = 16

NC = 2
NS = 16
NW = NC * NS

EDGES_PER_W = N_EDGES // NW
CHUNK = 80
N_CHUNKS = EDGES_PER_W // CHUNK
N_PAIRS = N_CHUNKS // 2


def _make_router():
    mesh = plsc.VectorSubcoreMesh(core_axis_name="c", subcore_axis_name="s")

    buf_set = dict(
        sidx=pltpu.VMEM((CHUNK,), jnp.int32),
        didx=pltpu.VMEM((CHUNK,), jnp.int32),
        srows=pltpu.VMEM((CHUNK, D_BLOCK), jnp.float32),
        drows=pltpu.VMEM((CHUNK, D_BLOCK), jnp.float32),
        attr=pltpu.VMEM((CHUNK, D_EDGE), jnp.float32),
        obuf=pltpu.VMEM((CHUNK, D_OUT), jnp.float32),
    )

    @functools.partial(
        pl.kernel,
        out_type=jax.ShapeDtypeStruct((N_EDGES, D_OUT), jnp.float32),
        mesh=mesh,
        scratch_types=(
            [v for v in buf_set.values()] * 2
            + [pltpu.SemaphoreType.DMA] * 8
        ),
    )
    def router(tbl_hbm, attr_hbm, eidx_hbm, out_hbm,
               sidxA, didxA, srowsA, drowsA, attrA, obufA,
               sidxB, didxB, srowsB, drowsB, attrB, obufB,
               isemA, isemB, asemA, asemB, gsemA, gsemB, osemA, osemB):
        wid = lax.axis_index("s") * NC + lax.axis_index("c")
        base = wid * EDGES_PER_W

        A = (sidxA, didxA, srowsA, drowsA, attrA, obufA, isemA, asemA, gsemA, osemA)
        B = (sidxB, didxB, srowsB, drowsB, attrB, obufB, isemB, asemB, gsemB, osemB)

        def off_of(c):
            return base + jnp.minimum(c, N_CHUNKS - 1) * CHUNK

        def issue_idx(c, s):
            off = off_of(c)
            pltpu.async_copy(eidx_hbm.at[pl.ds(off, CHUNK)], s[0], s[6])
            pltpu.async_copy(eidx_hbm.at[pl.ds(N_EDGES + off, CHUNK)], s[1], s[6])

        def wait_idx(s):
            pltpu.make_async_copy(eidx_hbm.at[pl.ds(0, CHUNK)], s[0], s[6]).wait()
            pltpu.make_async_copy(eidx_hbm.at[pl.ds(0, CHUNK)], s[1], s[6]).wait()

        def issue_attr(c, s):
            pltpu.async_copy(attr_hbm.at[pl.ds(off_of(c), CHUNK), :], s[4], s[7])

        def wait_attr(s):
            pltpu.make_async_copy(
                attr_hbm.at[pl.ds(0, CHUNK), :], s[4], s[7]).wait()

        def issue_gathers(s):
            pltpu.async_copy(tbl_hbm.at[s[0]], s[2], s[8])
            pltpu.async_copy(tbl_hbm.at[s[1]], s[3], s[8])

        def wait_gathers(s):
            pltpu.make_async_copy(tbl_hbm.at[s[0]], s[2], s[8]).wait()
            pltpu.make_async_copy(tbl_hbm.at[s[1]], s[3], s[8]).wait()

        def issue_store(c, s):
            pltpu.async_copy(s[5], out_hbm.at[pl.ds(base + c * CHUNK, CHUNK), :], s[9])

        def wait_store(s):
            pltpu.make_async_copy(
                s[5], out_hbm.at[pl.ds(0, CHUNK), :], s[9]).wait()

        def fill(s):
            srows, drows, attr, obuf = s[2], s[3], s[4], s[5]

            def row(r, c):
                obuf[r, pl.ds(0, D_EDGE)] = attr[r, :]
                for j in range(D_BLOCK // LANES):
                    obuf[r, pl.ds(D_EDGE + j * LANES, LANES)] = (
                        srows[r, pl.ds(j * LANES, LANES)])
                    obuf[r, pl.ds(D_EDGE + D_BLOCK + j * LANES, LANES)] = (
                        drows[r, pl.ds(j * LANES, LANES)])
                return c

            lax.fori_loop(0, CHUNK, row, 0)

        def half(k, c_now, c_pre, s_now, s_pre):
            wait_idx(s_pre)
            issue_gathers(s_pre)
            wait_gathers(s_now)
            issue_idx(c_pre, s_now)

            @pl.when(k > 0)
            def _():
                wait_store(s_now)

            wait_attr(s_now)
            fill(s_now)
            issue_store(c_now, s_now)
            issue_attr(c_pre, s_now)

        issue_idx(0, A)
        issue_attr(0, A)
        wait_idx(A)
        issue_gathers(A)
        issue_idx(1, B)
        issue_attr(1, B)

        def body(k, carry):
            cA = 2 * k
            half(k, cA, cA + 2, A, B)
            half(k, cA + 1, cA + 3, B, A)
            return carry

        lax.fori_loop(0, N_PAIRS, body, 0)

        wait_idx(B)
        wait_attr(B)
        wait_gathers(A)
        wait_store(A)
        wait_attr(A)
        fill(A)
        issue_store(N_CHUNKS - 1, A)
        wait_store(B)
        wait_store(A)

    return router


_router = _make_router()


def kernel(block_input, raw_input, edge_attr, edge_index):
    del raw_input
    eidx_flat = edge_index.astype(jnp.int32).reshape(-1)
    return _router(block_input, edge_attr, eidx_flat)
